# baseline (speedup 1.0000x reference)
import functools

import jax
import jax.numpy as jnp
from jax import lax
from jax.experimental import pallas as pl
from jax.experimental.pallas import tpu as pltpu
from jax.experimental.pallas import tpu_sc as plsc

H = 128


def _prep1_body(x_ref, ex_ref, wct_ref, bc_ref, wcat_ref, c_ref,
                cf_ref, a_ref, bp_ref):
    x = x_ref[0]
    cf = jnp.maximum(
        jnp.dot(x, wct_ref[...], preferred_element_type=jnp.float32)
        + bc_ref[...], 0.0)
    cf = cf * ex_ref[0]
    cf_ref[...] = cf
    ab = jnp.dot(cf, wcat_ref[...], preferred_element_type=jnp.float32)
    a_ref[...] = ab[:, :H]
    t = pl.program_id(1)
    bp_ref[...] = ab[:, H:] + c_ref[pl.ds(t, 1), :]


def _prep2_body(p_ref, wcat_ref, c_ref, cf_ref, a_ref, bp_ref):
    cf = p_ref[0] + p_ref[1]
    cf_ref[...] = cf
    ab = jnp.dot(cf, wcat_ref[...], preferred_element_type=jnp.float32)
    a_ref[...] = ab[:, :H]
    t = pl.program_id(1)
    bp_ref[...] = ab[:, H:] + c_ref[pl.ds(t, 1), :]


def _final_body(cf_ref, cf1_ref, p2_ref, ex_ref, wpt_ref, bp_ref,
                out_ref, acc_ref, es_ref):
    i = pl.program_id(0)
    nb = pl.num_programs(0)

    @pl.when(i == 0)
    def _():
        acc_ref[...] = jnp.zeros_like(acc_ref)
        es_ref[0, 0] = 0.0

    acc_ref[0:1, :] += jnp.sum(cf_ref[...], axis=0, keepdims=True)
    acc_ref[1:2, :] += jnp.sum(cf1_ref[...], axis=0, keepdims=True)
    acc_ref[2:3, :] += jnp.sum(p2_ref[0] + p2_ref[1], axis=0, keepdims=True)
    es_ref[0, 0] += jnp.sum(ex_ref[...])

    @pl.when(i == nb - 1)
    def _():
        s = jnp.concatenate(
            [acc_ref[0:1, :], acc_ref[1:2, :], acc_ref[2:3, :]], axis=1)
        s = s / es_ref[0, 0]
        out_ref[...] = jnp.maximum(
            jnp.dot(s, wpt_ref[...], preferred_element_type=jnp.float32)
            + bp_ref[...], 0.0)


def _make_prep1(n, fin, bs):
    nb = n // bs
    return pl.pallas_call(
        _prep1_body,
        grid=(nb, 4),
        in_specs=[
            pl.BlockSpec((1, bs, fin), lambda i, t: (0, i, 0)),
            pl.BlockSpec((1, bs, 1), lambda i, t: (0, i, 0)),
            pl.BlockSpec((fin, H), lambda i, t: (0, 0)),
            pl.BlockSpec((1, H), lambda i, t: (0, 0)),
            pl.BlockSpec((H, 2 * H), lambda i, t: (0, 0)),
            pl.BlockSpec((4, H), lambda i, t: (0, 0)),
        ],
        out_specs=[
            pl.BlockSpec((bs, H), lambda i, t: (i, 0)),
            pl.BlockSpec((bs, H), lambda i, t: (i, 0)),
            pl.BlockSpec((bs, H), lambda i, t: (t * nb + i, 0)),
        ],
        out_shape=[
            jax.ShapeDtypeStruct((n, H), jnp.float32),
            jax.ShapeDtypeStruct((n, H), jnp.float32),
            jax.ShapeDtypeStruct((4 * n, H), jnp.float32),
        ],
    )


def _make_prep2(n, n_pad, bs):
    nb = n // bs
    return pl.pallas_call(
        _prep2_body,
        grid=(nb, 4),
        in_specs=[
            pl.BlockSpec((2, bs, H), lambda i, t: (0, i, 0)),
            pl.BlockSpec((H, 2 * H), lambda i, t: (0, 0)),
            pl.BlockSpec((4, H), lambda i, t: (0, 0)),
        ],
        out_specs=[
            pl.BlockSpec((bs, H), lambda i, t: (i, 0)),
            pl.BlockSpec((bs, H), lambda i, t: (i, 0)),
            pl.BlockSpec((bs, H), lambda i, t: (t * nb + i, 0)),
        ],
        out_shape=[
            jax.ShapeDtypeStruct((n, H), jnp.float32),
            jax.ShapeDtypeStruct((n, H), jnp.float32),
            jax.ShapeDtypeStruct((4 * n, H), jnp.float32),
        ],
    )


def _make_final(n, n_pad, bs):
    nb = n // bs
    return pl.pallas_call(
        _final_body,
        grid=(nb,),
        in_specs=[
            pl.BlockSpec((bs, H), lambda i: (i, 0)),
            pl.BlockSpec((bs, H), lambda i: (i, 0)),
            pl.BlockSpec((2, bs, H), lambda i: (0, i, 0)),
            pl.BlockSpec((bs, 1), lambda i: (i, 0)),
            pl.BlockSpec((3 * H, H), lambda i: (0, 0)),
            pl.BlockSpec((1, H), lambda i: (0, 0)),
        ],
        out_specs=pl.BlockSpec((1, H), lambda i: (0, 0)),
        out_shape=jax.ShapeDtypeStruct((1, H), jnp.float32),
        scratch_shapes=[
            pltpu.VMEM((8, H), jnp.float32),
            pltpu.SMEM((1, 1), jnp.float32),
        ],
    )


def _make_edge_pass(n_pad, e):
    info = plsc.get_sparse_core_info()
    nc, ns = info.num_cores, info.num_subcores
    nw = nc * ns
    epw = e // nw
    K = 80
    nchunk = epw // K
    rps = n_pad // ns
    nz = rps // K

    mesh = plsc.VectorSubcoreMesh(core_axis_name="c", subcore_axis_name="s")

    @functools.partial(
        pl.kernel,
        out_type=jax.ShapeDtypeStruct((nc, n_pad, H), jnp.float32),
        mesh=mesh,
        scratch_types=[
            pltpu.VMEM((2, 2, K), jnp.int32),
            pltpu.VMEM((2, K, H), jnp.float32),
            pltpu.VMEM((2, K, H), jnp.float32),
            pltpu.VMEM_SHARED((n_pad, H), jnp.float32),
            pltpu.SemaphoreType.DMA,
            pltpu.SemaphoreType.DMA,
            pltpu.SemaphoreType.DMA,
            pltpu.SemaphoreType.DMA,
            pltpu.SemaphoreType.DMA,
            pltpu.SemaphoreType.DMA,
        ],
    )
    def edge_pass(a_hbm, bp_hbm, src_hbm, dstx_hbm, out_hbm,
                  idx_v, a_v, b_v, acc,
                  sem_a0, sem_b0, sem_a1, sem_b1, sem_s0, sem_s1):
        cid = lax.axis_index("c")
        sid = lax.axis_index("s")
        wid = sid * nc + cid
        sem_a = (sem_a0, sem_a1)
        sem_b = (sem_b0, sem_b1)
        sem_s = (sem_s0, sem_s1)

        def _zrow(i, _):
            for j in range(H // 16):
                a_v[0, i, pl.ds(j * 16, 16)] = jnp.zeros((16,), jnp.float32)
            return 0
        lax.fori_loop(0, K, _zrow, 0)
        for k in range(nz):
            pltpu.sync_copy(a_v.at[0], acc.at[pl.ds(sid * rps + k * K, K)])
        plsc.subcore_barrier()

        cbase = wid * nchunk
        clast = cbase + nchunk - 1

        def _issue(c, p):
            pltpu.sync_copy(src_hbm.at[pl.ds(c * K, K)], idx_v.at[p, 0])
            pltpu.sync_copy(dstx_hbm.at[pl.ds(c * K, K)], idx_v.at[p, 1])
            pltpu.async_copy(a_hbm.at[idx_v.at[p, 0]], a_v.at[p], sem_a[p])
            pltpu.async_copy(bp_hbm.at[idx_v.at[p, 1]], b_v.at[p], sem_b[p])

        def _wait(p):
            pltpu.make_async_copy(
                a_hbm.at[idx_v.at[p, 0]], a_v.at[p], sem_a[p]).wait()
            pltpu.make_async_copy(
                bp_hbm.at[idx_v.at[p, 1]], b_v.at[p], sem_b[p]).wait()

        def _compute(p):
            def _edge(ei, _):
                e0 = 2 * ei
                for dd in range(2):
                    for j in range(H // 16):
                        sl = pl.ds(j * 16, 16)
                        a_v[p, e0 + dd, sl] = jnp.maximum(
                            a_v[p, e0 + dd, sl] + b_v[p, e0 + dd, sl], 0.0)
                return 0
            lax.fori_loop(0, K // 2, _edge, 0)

        def _scatter_start(p):
            pltpu.async_copy(a_v.at[p], acc.at[idx_v.at[p, 0]], sem_s[p],
                             add=True)

        def _scatter_wait(p):
            pltpu.make_async_copy(a_v.at[p], acc.at[idx_v.at[p, 0]],
                                  sem_s[p]).wait()

        _issue(cbase, 0)
        _issue(cbase + 1, 1)

        def _pair(i, _):
            c0 = cbase + 2 * i
            _wait(0)
            _compute(0)
            _scatter_start(0)
            _wait(1)
            _scatter_wait(0)
            _issue(jnp.minimum(c0 + 2, clast), 0)
            _compute(1)
            _scatter_start(1)
            _scatter_wait(1)
            _issue(jnp.minimum(c0 + 3, clast), 1)
            return 0
        lax.fori_loop(0, nchunk // 2, _pair, 0)
        _wait(0)
        if nchunk % 2:
            _compute(0)
            _scatter_start(0)
            _scatter_wait(0)
        _wait(1)

        plsc.subcore_barrier()
        pltpu.sync_copy(acc.at[pl.ds(sid * rps, rps)],
                        out_hbm.at[cid, pl.ds(sid * rps, rps)])

    return edge_pass


def kernel(child_feats, child_exists, edge_type_onehot, edge_indices,
           W_child, b_child, W_ne0, b_ne0, W_ne1, b_ne1,
           W_parent, b_parent):
    n = child_feats.shape[1]
    fin = child_feats.shape[2]
    e = edge_indices.shape[1]

    ex = child_exists[0]
    ei = edge_indices[0].astype(jnp.int32)
    src = ei[:, 0]
    dst = ei[:, 1]
    et = jnp.argmax(edge_type_onehot[0], axis=1).astype(jnp.int32)
    dstx = et * n + dst

    def split_w(w, b):
        wcat = jnp.concatenate([w[:, :H].T, w[:, H:2 * H].T], axis=1)
        c = w[:, 2 * H:].T + b[None, :]
        return wcat, c

    wcat1, c1 = split_w(W_ne0, b_ne0)
    wcat2, c2 = split_w(W_ne1, b_ne1)

    bs = 1000
    n_pad = ((n + 1023) // 1024) * 1024
    prep1 = _make_prep1(n, fin, bs)
    prep2 = _make_prep2(n, n_pad, bs)
    final = _make_final(n, n_pad, bs)
    edge_pass = _make_edge_pass(n_pad, e)

    cf, a1, bp1 = prep1(child_feats, child_exists, W_child.T, b_child[None],
                        wcat1, c1)
    p1 = edge_pass(a1, bp1, src, dstx)
    cf1, a2, bp2 = prep2(p1, wcat2, c2)
    p2 = edge_pass(a2, bp2, src, dstx)
    return final(cf, cf1, p2, ex, W_parent.T, b_parent[None])


# speedup vs baseline: 1.0681x; 1.0681x over previous
import functools

import jax
import jax.numpy as jnp
from jax import lax
from jax.experimental import pallas as pl
from jax.experimental.pallas import tpu as pltpu
from jax.experimental.pallas import tpu_sc as plsc

H = 128


def _prep1_body(x_ref, ex_ref, wct_ref, bc_ref, wcat_ref, c_ref,
                cf_ref, a_ref, bp_ref):
    x = x_ref[0]
    cf = jnp.maximum(
        jnp.dot(x, wct_ref[...], preferred_element_type=jnp.float32)
        + bc_ref[...], 0.0)
    cf = cf * ex_ref[0]
    cf_ref[...] = cf
    ab = jnp.dot(cf, wcat_ref[...], preferred_element_type=jnp.float32)
    a_ref[...] = ab[:, :H]
    t = pl.program_id(1)
    bp_ref[...] = ab[:, H:] + c_ref[pl.ds(t, 1), :]


def _prep2_body(p_ref, wcat_ref, c_ref, cf_ref, a_ref, bp_ref):
    cf = p_ref[0] + p_ref[1]
    cf_ref[...] = cf
    ab = jnp.dot(cf, wcat_ref[...], preferred_element_type=jnp.float32)
    a_ref[...] = ab[:, :H]
    t = pl.program_id(1)
    bp_ref[...] = ab[:, H:] + c_ref[pl.ds(t, 1), :]


def _final_body(cf_ref, cf1_ref, p2_ref, ex_ref, wpt_ref, bp_ref,
                out_ref, acc_ref, es_ref):
    i = pl.program_id(0)
    nb = pl.num_programs(0)

    @pl.when(i == 0)
    def _():
        acc_ref[...] = jnp.zeros_like(acc_ref)
        es_ref[0, 0] = 0.0

    acc_ref[0:1, :] += jnp.sum(cf_ref[...], axis=0, keepdims=True)
    acc_ref[1:2, :] += jnp.sum(cf1_ref[...], axis=0, keepdims=True)
    acc_ref[2:3, :] += jnp.sum(p2_ref[0] + p2_ref[1], axis=0, keepdims=True)
    es_ref[0, 0] += jnp.sum(ex_ref[...])

    @pl.when(i == nb - 1)
    def _():
        s = jnp.concatenate(
            [acc_ref[0:1, :], acc_ref[1:2, :], acc_ref[2:3, :]], axis=1)
        s = s / es_ref[0, 0]
        out_ref[...] = jnp.maximum(
            jnp.dot(s, wpt_ref[...], preferred_element_type=jnp.float32)
            + bp_ref[...], 0.0)


def _make_prep1(n, fin, bs):
    nb = n // bs
    return pl.pallas_call(
        _prep1_body,
        grid=(nb, 4),
        in_specs=[
            pl.BlockSpec((1, bs, fin), lambda i, t: (0, i, 0)),
            pl.BlockSpec((1, bs, 1), lambda i, t: (0, i, 0)),
            pl.BlockSpec((fin, H), lambda i, t: (0, 0)),
            pl.BlockSpec((1, H), lambda i, t: (0, 0)),
            pl.BlockSpec((H, 2 * H), lambda i, t: (0, 0)),
            pl.BlockSpec((4, H), lambda i, t: (0, 0)),
        ],
        out_specs=[
            pl.BlockSpec((bs, H), lambda i, t: (i, 0)),
            pl.BlockSpec((bs, H), lambda i, t: (i, 0)),
            pl.BlockSpec((bs, H), lambda i, t: (t * nb + i, 0)),
        ],
        out_shape=[
            jax.ShapeDtypeStruct((n, H), jnp.float32),
            jax.ShapeDtypeStruct((n, H), jnp.float32),
            jax.ShapeDtypeStruct((4 * n, H), jnp.float32),
        ],
    )


def _make_prep2(n, n_pad, bs):
    nb = n // bs
    return pl.pallas_call(
        _prep2_body,
        grid=(nb, 4),
        in_specs=[
            pl.BlockSpec((2, bs, H), lambda i, t: (0, i, 0)),
            pl.BlockSpec((H, 2 * H), lambda i, t: (0, 0)),
            pl.BlockSpec((4, H), lambda i, t: (0, 0)),
        ],
        out_specs=[
            pl.BlockSpec((bs, H), lambda i, t: (i, 0)),
            pl.BlockSpec((bs, H), lambda i, t: (i, 0)),
            pl.BlockSpec((bs, H), lambda i, t: (t * nb + i, 0)),
        ],
        out_shape=[
            jax.ShapeDtypeStruct((n, H), jnp.float32),
            jax.ShapeDtypeStruct((n, H), jnp.float32),
            jax.ShapeDtypeStruct((4 * n, H), jnp.float32),
        ],
    )


def _make_final(n, n_pad, bs):
    nb = n // bs
    return pl.pallas_call(
        _final_body,
        grid=(nb,),
        in_specs=[
            pl.BlockSpec((bs, H), lambda i: (i, 0)),
            pl.BlockSpec((bs, H), lambda i: (i, 0)),
            pl.BlockSpec((2, bs, H), lambda i: (0, i, 0)),
            pl.BlockSpec((bs, 1), lambda i: (i, 0)),
            pl.BlockSpec((3 * H, H), lambda i: (0, 0)),
            pl.BlockSpec((1, H), lambda i: (0, 0)),
        ],
        out_specs=pl.BlockSpec((1, H), lambda i: (0, 0)),
        out_shape=jax.ShapeDtypeStruct((1, H), jnp.float32),
        scratch_shapes=[
            pltpu.VMEM((8, H), jnp.float32),
            pltpu.SMEM((1, 1), jnp.float32),
        ],
    )


def _make_edge_pass(n_pad, e):
    info = plsc.get_sparse_core_info()
    nc, ns = info.num_cores, info.num_subcores
    nw = nc * ns
    epw = e // nw
    K = 80
    nchunk = epw // K
    rps = n_pad // ns
    nz = rps // K

    mesh = plsc.VectorSubcoreMesh(core_axis_name="c", subcore_axis_name="s")

    @functools.partial(
        pl.kernel,
        out_type=jax.ShapeDtypeStruct((nc, n_pad, H), jnp.float32),
        mesh=mesh,
        scratch_types=[
            pltpu.VMEM((2, 2, K), jnp.int32),
            pltpu.VMEM((2, K, H), jnp.float32),
            pltpu.VMEM((2, K, H), jnp.float32),
            pltpu.VMEM_SHARED((n_pad, H), jnp.float32),
            pltpu.SemaphoreType.DMA,
            pltpu.SemaphoreType.DMA,
            pltpu.SemaphoreType.DMA,
            pltpu.SemaphoreType.DMA,
        ],
    )
    def edge_pass(a_hbm, bp_hbm, src_hbm, dstx_hbm, out_hbm,
                  idx_v, a_v, b_v, acc,
                  sem_a0, sem_b0, sem_a1, sem_b1):
        cid = lax.axis_index("c")
        sid = lax.axis_index("s")
        wid = sid * nc + cid
        sem_a = (sem_a0, sem_a1)
        sem_b = (sem_b0, sem_b1)

        def _zrow(i, _):
            for j in range(H // 16):
                a_v[0, i, pl.ds(j * 16, 16)] = jnp.zeros((16,), jnp.float32)
            return 0
        lax.fori_loop(0, K, _zrow, 0)
        for k in range(nz):
            pltpu.sync_copy(a_v.at[0], acc.at[pl.ds(sid * rps + k * K, K)])
        plsc.subcore_barrier()

        cbase = wid * nchunk
        clast = cbase + nchunk - 1

        def _issue(c, p):
            pltpu.sync_copy(src_hbm.at[pl.ds(c * K, K)], idx_v.at[p, 0])
            pltpu.sync_copy(dstx_hbm.at[pl.ds(c * K, K)], idx_v.at[p, 1])
            pltpu.async_copy(a_hbm.at[idx_v.at[p, 0]], a_v.at[p], sem_a[p])
            pltpu.async_copy(bp_hbm.at[idx_v.at[p, 1]], b_v.at[p], sem_b[p])

        def _wait(p):
            pltpu.make_async_copy(
                a_hbm.at[idx_v.at[p, 0]], a_v.at[p], sem_a[p]).wait()
            pltpu.make_async_copy(
                bp_hbm.at[idx_v.at[p, 1]], b_v.at[p], sem_b[p]).wait()

        def _compute_scatter(p):
            def _edge(ei, _):
                for j in range(H // 16):
                    sl = pl.ds(j * 16, 16)
                    a_v[p, ei, sl] = jnp.maximum(
                        a_v[p, ei, sl] + b_v[p, ei, sl], 0.0)
                return 0
            lax.fori_loop(0, K, _edge, 0)
            pltpu.sync_copy(a_v.at[p], acc.at[idx_v.at[p, 0]], add=True)

        _issue(cbase, 0)
        _issue(cbase + 1, 1)

        def _pair(i, _):
            c0 = cbase + 2 * i
            _wait(0)
            _compute_scatter(0)
            _issue(jnp.minimum(c0 + 2, clast), 0)
            _wait(1)
            _compute_scatter(1)
            _issue(jnp.minimum(c0 + 3, clast), 1)
            return 0
        lax.fori_loop(0, nchunk // 2, _pair, 0)
        _wait(0)
        if nchunk % 2:
            _compute_scatter(0)
        _wait(1)

        plsc.subcore_barrier()
        pltpu.sync_copy(acc.at[pl.ds(sid * rps, rps)],
                        out_hbm.at[cid, pl.ds(sid * rps, rps)])

    return edge_pass


def kernel(child_feats, child_exists, edge_type_onehot, edge_indices,
           W_child, b_child, W_ne0, b_ne0, W_ne1, b_ne1,
           W_parent, b_parent):
    n = child_feats.shape[1]
    fin = child_feats.shape[2]
    e = edge_indices.shape[1]

    ex = child_exists[0]
    ei = edge_indices[0].astype(jnp.int32)
    src = ei[:, 0]
    dst = ei[:, 1]
    et = jnp.argmax(edge_type_onehot[0], axis=1).astype(jnp.int32)
    dstx = et * n + dst

    def split_w(w, b):
        wcat = jnp.concatenate([w[:, :H].T, w[:, H:2 * H].T], axis=1)
        c = w[:, 2 * H:].T + b[None, :]
        return wcat, c

    wcat1, c1 = split_w(W_ne0, b_ne0)
    wcat2, c2 = split_w(W_ne1, b_ne1)

    bs = 1000
    n_pad = ((n + 1023) // 1024) * 1024
    prep1 = _make_prep1(n, fin, bs)
    prep2 = _make_prep2(n, n_pad, bs)
    final = _make_final(n, n_pad, bs)
    edge_pass = _make_edge_pass(n_pad, e)

    cf, a1, bp1 = prep1(child_feats, child_exists, W_child.T, b_child[None],
                        wcat1, c1)
    p1 = edge_pass(a1, bp1, src, dstx)
    cf1, a2, bp2 = prep2(p1, wcat2, c2)
    p2 = edge_pass(a2, bp2, src, dstx)
    return final(cf, cf1, p2, ex, W_parent.T, b_parent[None])


# speedup vs baseline: 1.3467x; 1.2608x over previous
import functools

import jax
import jax.numpy as jnp
from jax import lax
from jax.experimental import pallas as pl
from jax.experimental.pallas import tpu as pltpu
from jax.experimental.pallas import tpu_sc as plsc

H = 128


def _prep1_body(x_ref, ex_ref, wct_ref, bc_ref, wcat_ref, c_ref,
                cf_ref, a_ref, bp_ref):
    x = x_ref[0]
    cf = jnp.maximum(
        jnp.dot(x, wct_ref[...], preferred_element_type=jnp.float32)
        + bc_ref[...], 0.0)
    cf = cf * ex_ref[0]
    cf_ref[...] = cf
    ab = jnp.dot(cf, wcat_ref[...], preferred_element_type=jnp.float32)
    a_ref[...] = ab[:, :H]
    t = pl.program_id(1)
    bp_ref[...] = ab[:, H:] + c_ref[pl.ds(t, 1), :]


def _prep2_body(p_ref, wcat_ref, c_ref, cf_ref, a_ref, bp_ref):
    cf = p_ref[0] + p_ref[1]
    cf_ref[...] = cf
    ab = jnp.dot(cf, wcat_ref[...], preferred_element_type=jnp.float32)
    a_ref[...] = ab[:, :H]
    t = pl.program_id(1)
    bp_ref[...] = ab[:, H:] + c_ref[pl.ds(t, 1), :]


def _final_body(cf_ref, cf1_ref, p2_ref, ex_ref, wpt_ref, bp_ref,
                out_ref, acc_ref, es_ref):
    i = pl.program_id(0)
    nb = pl.num_programs(0)

    @pl.when(i == 0)
    def _():
        acc_ref[...] = jnp.zeros_like(acc_ref)
        es_ref[0, 0] = 0.0

    acc_ref[0:1, :] += jnp.sum(cf_ref[...], axis=0, keepdims=True)
    acc_ref[1:2, :] += jnp.sum(cf1_ref[...], axis=0, keepdims=True)
    acc_ref[2:3, :] += jnp.sum(p2_ref[0] + p2_ref[1], axis=0, keepdims=True)
    es_ref[0, 0] += jnp.sum(ex_ref[...])

    @pl.when(i == nb - 1)
    def _():
        s = jnp.concatenate(
            [acc_ref[0:1, :], acc_ref[1:2, :], acc_ref[2:3, :]], axis=1)
        s = s / es_ref[0, 0]
        out_ref[...] = jnp.maximum(
            jnp.dot(s, wpt_ref[...], preferred_element_type=jnp.float32)
            + bp_ref[...], 0.0)


def _make_prep1(n, fin, bs):
    nb = n // bs
    return pl.pallas_call(
        _prep1_body,
        grid=(nb, 4),
        in_specs=[
            pl.BlockSpec((1, bs, fin), lambda i, t: (0, i, 0)),
            pl.BlockSpec((1, bs, 1), lambda i, t: (0, i, 0)),
            pl.BlockSpec((fin, H), lambda i, t: (0, 0)),
            pl.BlockSpec((1, H), lambda i, t: (0, 0)),
            pl.BlockSpec((H, 2 * H), lambda i, t: (0, 0)),
            pl.BlockSpec((4, H), lambda i, t: (0, 0)),
        ],
        out_specs=[
            pl.BlockSpec((bs, H), lambda i, t: (i, 0)),
            pl.BlockSpec((bs, H), lambda i, t: (i, 0)),
            pl.BlockSpec((bs, H), lambda i, t: (t * nb + i, 0)),
        ],
        out_shape=[
            jax.ShapeDtypeStruct((n, H), jnp.float32),
            jax.ShapeDtypeStruct((n, H), jnp.float32),
            jax.ShapeDtypeStruct((4 * n, H), jnp.float32),
        ],
    )


def _make_prep2(n, n_pad, bs):
    nb = n // bs
    return pl.pallas_call(
        _prep2_body,
        grid=(nb, 4),
        in_specs=[
            pl.BlockSpec((2, bs, H), lambda i, t: (0, i, 0)),
            pl.BlockSpec((H, 2 * H), lambda i, t: (0, 0)),
            pl.BlockSpec((4, H), lambda i, t: (0, 0)),
        ],
        out_specs=[
            pl.BlockSpec((bs, H), lambda i, t: (i, 0)),
            pl.BlockSpec((bs, H), lambda i, t: (i, 0)),
            pl.BlockSpec((bs, H), lambda i, t: (t * nb + i, 0)),
        ],
        out_shape=[
            jax.ShapeDtypeStruct((n, H), jnp.float32),
            jax.ShapeDtypeStruct((n, H), jnp.float32),
            jax.ShapeDtypeStruct((4 * n, H), jnp.float32),
        ],
    )


def _make_final(n, n_pad, bs):
    nb = n // bs
    return pl.pallas_call(
        _final_body,
        grid=(nb,),
        in_specs=[
            pl.BlockSpec((bs, H), lambda i: (i, 0)),
            pl.BlockSpec((bs, H), lambda i: (i, 0)),
            pl.BlockSpec((2, bs, H), lambda i: (0, i, 0)),
            pl.BlockSpec((bs, 1), lambda i: (i, 0)),
            pl.BlockSpec((3 * H, H), lambda i: (0, 0)),
            pl.BlockSpec((1, H), lambda i: (0, 0)),
        ],
        out_specs=pl.BlockSpec((1, H), lambda i: (0, 0)),
        out_shape=jax.ShapeDtypeStruct((1, H), jnp.float32),
        scratch_shapes=[
            pltpu.VMEM((8, H), jnp.float32),
            pltpu.SMEM((1, 1), jnp.float32),
        ],
    )


def _make_edge_pass(n_pad, e):
    info = plsc.get_sparse_core_info()
    nc, ns = info.num_cores, info.num_subcores
    nw = nc * ns
    epw = e // nw
    K = 80
    nchunk = epw // K
    rps = n_pad // ns
    nz = rps // K

    mesh = plsc.VectorSubcoreMesh(core_axis_name="c", subcore_axis_name="s")

    @functools.partial(
        pl.kernel,
        out_type=jax.ShapeDtypeStruct((nc, n_pad, H), jnp.float32),
        mesh=mesh,
        scratch_types=[
            pltpu.VMEM((2, 2, 2, K), jnp.int32),
            pltpu.VMEM((2, K, H), jnp.float32),
            pltpu.VMEM((2, K, H), jnp.float32),
            pltpu.VMEM_SHARED((n_pad, H), jnp.float32),
            pltpu.SemaphoreType.DMA,
            pltpu.SemaphoreType.DMA,
            pltpu.SemaphoreType.DMA,
            pltpu.SemaphoreType.DMA,
            pltpu.SemaphoreType.DMA,
            pltpu.SemaphoreType.DMA,
        ],
    )
    def edge_pass(a_hbm, bp_hbm, src_hbm, dstx_hbm, out_hbm,
                  idx_v, a_v, b_v, acc,
                  sem_a0, sem_b0, sem_a1, sem_b1, sem_i0, sem_i1):
        cid = lax.axis_index("c")
        sid = lax.axis_index("s")
        wid = sid * nc + cid
        sem_a = (sem_a0, sem_a1)
        sem_b = (sem_b0, sem_b1)
        sem_i = (sem_i0, sem_i1)

        def _zrow(i, _):
            for j in range(H // 16):
                a_v[0, i, pl.ds(j * 16, 16)] = jnp.zeros((16,), jnp.float32)
            return 0
        lax.fori_loop(0, K, _zrow, 0)
        for k in range(nz):
            pltpu.sync_copy(a_v.at[0], acc.at[pl.ds(sid * rps + k * K, K)])
        plsc.subcore_barrier()

        ebase = wid * epw

        def _issue_idx(p, pp):
            off = ebase + jnp.minimum(p * 2 * K, epw - 2 * K)
            pltpu.async_copy(src_hbm.at[pl.ds(off, K)],
                             idx_v.at[pp, 0, 0], sem_i[pp])
            pltpu.async_copy(src_hbm.at[pl.ds(off + K, K)],
                             idx_v.at[pp, 0, 1], sem_i[pp])
            pltpu.async_copy(dstx_hbm.at[pl.ds(off, K)],
                             idx_v.at[pp, 1, 0], sem_i[pp])
            pltpu.async_copy(dstx_hbm.at[pl.ds(off + K, K)],
                             idx_v.at[pp, 1, 1], sem_i[pp])

        def _wait_idx(pp):
            for sd in range(2):
                for h in range(2):
                    pltpu.make_async_copy(
                        src_hbm.at[pl.ds(0, K)],
                        idx_v.at[pp, sd, h], sem_i[pp]).wait()

        def _issue_g(dp, pp, h):
            pltpu.async_copy(a_hbm.at[idx_v.at[pp, 0, h]],
                             a_v.at[dp], sem_a[dp])
            pltpu.async_copy(bp_hbm.at[idx_v.at[pp, 1, h]],
                             b_v.at[dp], sem_b[dp])

        def _wait_g(dp):
            pltpu.make_async_copy(
                a_hbm.at[idx_v.at[0, 0, 0]], a_v.at[dp], sem_a[dp]).wait()
            pltpu.make_async_copy(
                bp_hbm.at[idx_v.at[0, 1, 0]], b_v.at[dp], sem_b[dp]).wait()

        def _cs(dp, pp, h):
            def _edge(ei, _):
                for j in range(H // 16):
                    sl = pl.ds(j * 16, 16)
                    a_v[dp, ei, sl] = jnp.maximum(
                        a_v[dp, ei, sl] + b_v[dp, ei, sl], 0.0)
                return 0
            lax.fori_loop(0, K, _edge, 0)
            pltpu.sync_copy(a_v.at[dp], acc.at[idx_v.at[pp, 0, h]], add=True)

        _issue_idx(jnp.int32(0), 0)
        _issue_idx(jnp.int32(1), 1)
        _wait_idx(0)
        _issue_g(0, 0, 0)
        _issue_g(1, 0, 1)

        def _quad(i, _):
            _wait_idx(1)
            _wait_g(0)
            _cs(0, 0, 0)
            _issue_g(0, 1, 0)
            _wait_g(1)
            _cs(1, 0, 1)
            _issue_idx(2 * i + 2, 0)
            _issue_g(1, 1, 1)
            _wait_g(0)
            _cs(0, 1, 0)
            _wait_idx(0)
            _issue_g(0, 0, 0)
            _wait_g(1)
            _cs(1, 1, 1)
            _issue_idx(2 * i + 3, 1)
            _issue_g(1, 0, 1)
            return 0
        lax.fori_loop(0, nchunk // 4, _quad, 0)

        _wait_g(0)
        _wait_g(1)
        _cs(1, 0, 1)
        _wait_idx(1)

        plsc.subcore_barrier()
        pltpu.sync_copy(acc.at[pl.ds(sid * rps, rps)],
                        out_hbm.at[cid, pl.ds(sid * rps, rps)])

    return edge_pass


def kernel(child_feats, child_exists, edge_type_onehot, edge_indices,
           W_child, b_child, W_ne0, b_ne0, W_ne1, b_ne1,
           W_parent, b_parent):
    n = child_feats.shape[1]
    fin = child_feats.shape[2]
    e = edge_indices.shape[1]

    ex = child_exists[0]
    ei = edge_indices[0].astype(jnp.int32)
    src = ei[:, 0]
    dst = ei[:, 1]
    et = jnp.argmax(edge_type_onehot[0], axis=1).astype(jnp.int32)
    dstx = et * n + dst

    def split_w(w, b):
        wcat = jnp.concatenate([w[:, :H].T, w[:, H:2 * H].T], axis=1)
        c = w[:, 2 * H:].T + b[None, :]
        return wcat, c

    wcat1, c1 = split_w(W_ne0, b_ne0)
    wcat2, c2 = split_w(W_ne1, b_ne1)

    bs = 1000
    n_pad = ((n + 1023) // 1024) * 1024
    prep1 = _make_prep1(n, fin, bs)
    prep2 = _make_prep2(n, n_pad, bs)
    final = _make_final(n, n_pad, bs)
    edge_pass = _make_edge_pass(n_pad, e)

    cf, a1, bp1 = prep1(child_feats, child_exists, W_child.T, b_child[None],
                        wcat1, c1)
    p1 = edge_pass(a1, bp1, src, dstx)
    cf1, a2, bp2 = prep2(p1, wcat2, c2)
    p2 = edge_pass(a2, bp2, src, dstx)
    return final(cf, cf1, p2, ex, W_parent.T, b_parent[None])
